# baseline (speedup 1.0000x reference)
import functools

import jax
import jax.numpy as jnp
from jax import lax
from jax.experimental import pallas as pl
from jax.experimental.pallas import tpu as pltpu
from jax.experimental.pallas import tpu_sc as plsc

INDIM = 16384
OUTDIM = 16384
NNZ = 262144
BATCH = 64

NC = 2
NS = 16
L = 16

HB = BATCH // NC
K = 512
EPT = NNZ // NS
NCHUNK = EPT // K
ROWS_PT = OUTDIM // NS
FEAT_PT = INDIM // NS
IDXW = 128
NSUB = K // IDXW
HFEAT = FEAT_PT // 2


@functools.cache
def _build_sc_expander():
    return functools.partial(
        pl.kernel,
        out_type=(
            jax.ShapeDtypeStruct((BATCH, OUTDIM), jnp.float32),
            jax.ShapeDtypeStruct((NC * INDIM, HB), jnp.float32),
        ),
        mesh=plsc.VectorSubcoreMesh(
            core_axis_name="c", subcore_axis_name="s",
            num_cores=NC, num_subcores=NS,
        ),
        compiler_params=pltpu.CompilerParams(
            use_tc_tiling_on_sc=False, needs_layout_passes=False
        ),
        scratch_types=[
            pltpu.VMEM_SHARED((OUTDIM, HB), jnp.float32),
            pltpu.VMEM((4, NSUB, IDXW), jnp.int32),
            pltpu.VMEM((4, NSUB, IDXW), jnp.int32),
            pltpu.VMEM((4, K), jnp.float32),
            pltpu.VMEM((K, HB), jnp.float32),
            pltpu.VMEM((K, HB), jnp.float32),
            pltpu.VMEM((HB, HFEAT), jnp.float32),
            pltpu.VMEM((ROWS_PT,), jnp.float32),
            pltpu.SemaphoreType.DMA,
            pltpu.SemaphoreType.DMA,
            pltpu.SemaphoreType.DMA,
            pltpu.SemaphoreType.DMA,
            pltpu.SemaphoreType.DMA,
            pltpu.SemaphoreType.DMA,
            pltpu.SemaphoreType.DMA,
            pltpu.SemaphoreType.DMA,
        ],
    )(_sc_expander_body)


def _sc_expander_body(x, colm, rowm, w, bias, out, xflat, acc_sh, colb, rowb,
                      wb, vals0, vals1, tbuf, bias_v, isem0, isem1, isem2,
                      isem3, gsem0, gsem1, ssem0, ssem1):
    cid = lax.axis_index("c")
    sid = lax.axis_index("s")
    vals = (vals0, vals1)
    isem = (isem0, isem1, isem2, isem3)
    gsem = (gsem0, gsem1)
    ssem = (ssem0, ssem1)
    iota_l = lax.iota(jnp.int32, L)

    b0 = cid * HB
    f0 = pl.multiple_of(sid * FEAT_PT, FEAT_PT)
    r0 = pl.multiple_of(sid * ROWS_PT, ROWS_PT)

    for h in range(FEAT_PT // HFEAT):
        fh = f0 + h * HFEAT
        for b in range(HB):
            pltpu.async_copy(x.at[b0 + b, pl.ds(fh, HFEAT)], tbuf.at[b],
                             isem0)
        for b in range(HB):
            pltpu.make_async_copy(x.at[b0 + b, pl.ds(fh, HFEAT)], tbuf.at[b],
                                  isem0).wait()
        for b in range(HB):
            cb = jnp.full((L,), b, jnp.int32)

            @plsc.parallel_loop(0, HFEAT // L, unroll=2)
            def _(g):
                fi = jnp.broadcast_to(g * L, (L,)) + iota_l
                plsc.store_scatter(vals0, [fi, cb], tbuf[b, pl.ds(g * L, L)])

        pltpu.sync_copy(vals0, xflat.at[pl.ds(cid * INDIM + fh, HFEAT)])

    pltpu.sync_copy(bias.at[pl.ds(r0, ROWS_PT)], bias_v)
    for rb in range(ROWS_PT // K):

        @plsc.parallel_loop(0, K // L, unroll=1)
        def _(g):
            b16 = bias_v[pl.ds(rb * K + g * L, L)]
            for i in range(L):
                o = g * L + i
                bv = jnp.broadcast_to(b16[i], (L,))
                vals0[o, pl.ds(0, L)] = bv
                vals0[o, pl.ds(L, L)] = bv

        pltpu.sync_copy(vals0, acc_sh.at[pl.ds(r0 + rb * K, K)])

    plsc.subcore_barrier()

    ebase = pl.multiple_of(sid * EPT, EPT)
    cshift = cid * INDIM

    def issue_idx(ck, s):
        e0 = pl.multiple_of(ebase + ck * K, K)
        m0 = pl.multiple_of(e0 // IDXW, NSUB)
        pltpu.async_copy(colm.at[pl.ds(m0, NSUB)], colb.at[s], isem[s])
        pltpu.async_copy(rowm.at[pl.ds(m0, NSUB)], rowb.at[s], isem[s])
        pltpu.async_copy(w.at[pl.ds(e0, K)], wb.at[s], isem[s])

    def wait_idx(ck, s):
        e0 = pl.multiple_of(ebase + ck * K, K)
        m0 = pl.multiple_of(e0 // IDXW, NSUB)
        pltpu.make_async_copy(colm.at[pl.ds(m0, NSUB)], colb.at[s],
                              isem[s]).wait()
        pltpu.make_async_copy(rowm.at[pl.ds(m0, NSUB)], rowb.at[s],
                              isem[s]).wait()
        pltpu.make_async_copy(w.at[pl.ds(e0, K)], wb.at[s], isem[s]).wait()
        for j in range(NSUB):
            for kk in range(IDXW // L):
                sl = pl.ds(kk * L, L)
                colb[s, j, sl] = colb[s, j, sl] + cshift

    def issue_gathers(s, vp):
        for j in range(NSUB):
            pltpu.async_copy(
                xflat.at[colb.at[s, j]],
                vals[vp].at[pl.ds(j * IDXW, IDXW)],
                gsem[vp],
            )

    def wait_gathers(s, vp):
        for j in range(NSUB):
            pltpu.make_async_copy(
                xflat.at[colb.at[s, j]],
                vals[vp].at[pl.ds(j * IDXW, IDXW)],
                gsem[vp],
            ).wait()

    def issue_scatters(s, vp):
        for j in range(NSUB):
            pltpu.async_copy(
                vals[vp].at[pl.ds(j * IDXW, IDXW)],
                acc_sh.at[rowb.at[s, j]],
                ssem[vp],
                add=True,
            )

    def wait_scatters(s, vp):
        for j in range(NSUB):
            pltpu.make_async_copy(
                vals[vp].at[pl.ds(j * IDXW, IDXW)],
                acc_sh.at[rowb.at[s, j]],
                ssem[vp],
            ).wait()

    def scale(s, vp):
        vb = vals[vp]

        @plsc.parallel_loop(0, K // L, unroll=1)
        def _(g):
            w16 = wb[s, pl.ds(g * L, L)]
            for i in range(L):
                e = g * L + i
                wv = jnp.broadcast_to(w16[i], (L,))
                vb[e, pl.ds(0, L)] = vb[e, pl.ds(0, L)] * wv
                vb[e, pl.ds(L, L)] = vb[e, pl.ds(L, L)] * wv

    issue_idx(0, 0)
    issue_idx(1, 1)
    wait_idx(0, 0)
    issue_gathers(0, 0)

    @pl.loop(0, NCHUNK, step=4)
    def _(c):
        for m in range(4):
            cm = c + m
            sm = m
            vp = m & 1

            @pl.when(cm + 2 < NCHUNK)
            def _():
                issue_idx(cm + 2, (m + 2) % 4)

            wait_gathers(sm, vp)

            @pl.when(cm >= 1)
            def _():
                wait_scatters((m + 3) % 4, 1 - vp)

            @pl.when(cm + 1 < NCHUNK)
            def _():
                wait_idx(cm + 1, (m + 1) % 4)
                issue_gathers((m + 1) % 4, 1 - vp)

            scale(sm, vp)
            issue_scatters(sm, vp)

    wait_scatters(3, 1)

    plsc.subcore_barrier()
    for h in range(ROWS_PT // K):
        rh = r0 + h * K
        pltpu.sync_copy(acc_sh.at[pl.ds(rh, K)], vals0)
        for b in range(HB):
            cb = jnp.full((L,), b, jnp.int32)

            @plsc.parallel_loop(0, K // L, unroll=2)
            def _(g):
                fi = jnp.broadcast_to(g * L, (L,)) + iota_l
                tbuf[b, pl.ds(g * L, L)] = plsc.load_gather(vals0, [fi, cb])

        for b in range(HB):
            pltpu.async_copy(tbuf.at[b], out.at[b0 + b, pl.ds(rh, K)], isem0)
        for b in range(HB):
            pltpu.make_async_copy(tbuf.at[b], out.at[b0 + b, pl.ds(rh, K)],
                                  isem0).wait()


def kernel(x, row, col, nnz_weight, bias):
    colm = col.reshape(NNZ // IDXW, IDXW)
    rowm = row.reshape(NNZ // IDXW, IDXW)
    out, _ = _build_sc_expander()(x, colm, rowm, nnz_weight, bias)
    return out


# speedup vs baseline: 1.1770x; 1.1770x over previous
import functools

import jax
import jax.numpy as jnp
from jax import lax
from jax.experimental import pallas as pl
from jax.experimental.pallas import tpu as pltpu
from jax.experimental.pallas import tpu_sc as plsc

INDIM = 16384
OUTDIM = 16384
NNZ = 262144
BATCH = 64

NC = 2
NS = 16
L = 16

HB = BATCH // NC
K = 512
EPT = NNZ // NS
NCHUNK = EPT // K
ROWS_PT = OUTDIM // NS
IDXW = 128
NSUB = K // IDXW
GBYTES = IDXW * HB * 4


@functools.cache
def _build_sc_expander():
    return functools.partial(
        pl.kernel,
        out_type=jax.ShapeDtypeStruct((NC, OUTDIM, HB), jnp.bfloat16),
        mesh=plsc.VectorSubcoreMesh(
            core_axis_name="c", subcore_axis_name="s",
            num_cores=NC, num_subcores=NS,
        ),
        compiler_params=pltpu.CompilerParams(
            use_tc_tiling_on_sc=False, needs_layout_passes=False
        ),
        scratch_types=[
            pltpu.VMEM_SHARED((OUTDIM, HB), jnp.bfloat16),
            pltpu.VMEM((EPT // IDXW, IDXW), jnp.int32),
            pltpu.VMEM((EPT // IDXW, IDXW), jnp.int32),
            pltpu.VMEM((EPT,), jnp.float32),
            pltpu.VMEM((K, HB), jnp.bfloat16),
            pltpu.VMEM((K, HB), jnp.bfloat16),
            pltpu.VMEM((ROWS_PT,), jnp.float32),
            pltpu.SemaphoreType.DMA,
            pltpu.SemaphoreType.DMA,
            pltpu.SemaphoreType.DMA,
            pltpu.SemaphoreType.DMA,
            pltpu.SemaphoreType.DMA,
        ],
    )(_sc_expander_body)


def _sc_expander_body(xflat, colm, rowm, w, bias, out, acc_sh, colt, rowt,
                      wt, vals0, vals1, bias_v, isem, gsem0, gsem1, ssem0,
                      ssem1):
    cid = lax.axis_index("c")
    sid = lax.axis_index("s")
    vals = (vals0, vals1)
    gsem = (gsem0, gsem1)
    ssem = (ssem0, ssem1)

    mrow0 = pl.multiple_of(sid * (EPT // IDXW), EPT // IDXW)
    e0 = pl.multiple_of(sid * EPT, EPT)
    pltpu.async_copy(colm.at[pl.ds(mrow0, EPT // IDXW)], colt, isem)
    pltpu.async_copy(rowm.at[pl.ds(mrow0, EPT // IDXW)], rowt, isem)
    pltpu.async_copy(w.at[pl.ds(e0, EPT)], wt, isem)

    r0 = pl.multiple_of(sid * ROWS_PT, ROWS_PT)
    pltpu.sync_copy(bias.at[pl.ds(r0, ROWS_PT)], bias_v)
    for rb in range(ROWS_PT // K):

        @plsc.parallel_loop(0, K // L, unroll=1)
        def _(g):
            b16 = bias_v[pl.ds(rb * K + g * L, L)]
            for i in range(L):
                o = g * L + i
                bvf = jnp.broadcast_to(b16[i], (L,))
                vals0[o, pl.ds(0, 2 * L)] = plsc.pack(
                    bvf, bvf, format=plsc.PackFormat.INTERLEAVED
                )

        pltpu.sync_copy(vals0, acc_sh.at[pl.ds(r0 + rb * K, K)])

    pltpu.make_async_copy(colm.at[pl.ds(mrow0, EPT // IDXW)], colt, isem).wait()
    pltpu.make_async_copy(rowm.at[pl.ds(mrow0, EPT // IDXW)], rowt, isem).wait()
    pltpu.make_async_copy(w.at[pl.ds(e0, EPT)], wt, isem).wait()

    cshift = cid * INDIM

    @plsc.parallel_loop(0, EPT // IDXW, unroll=2)
    def _(j):
        for kk in range(IDXW // L):
            sl = pl.ds(kk * L, L)
            colt[j, sl] = colt[j, sl] + cshift

    plsc.subcore_barrier()

    def issue_gathers(ck, b):
        for j in range(NSUB):
            pltpu.async_copy(
                xflat.at[colt.at[ck * NSUB + j]],
                vals[b].at[pl.ds(j * IDXW, IDXW)],
                gsem[b],
            )

    def wait_gathers(ck, b):
        for j in range(NSUB):
            pltpu.make_async_copy(
                xflat.at[colt.at[ck * NSUB + j]],
                vals[b].at[pl.ds(j * IDXW, IDXW)],
                gsem[b],
            ).wait()

    def issue_scatters(ck, b):
        for j in range(NSUB):
            pltpu.async_copy(
                vals[b].at[pl.ds(j * IDXW, IDXW)],
                acc_sh.at[rowt.at[ck * NSUB + j]],
                ssem[b],
                add=True,
            )

    def wait_scatters(ck, b):
        for j in range(NSUB):
            pltpu.make_async_copy(
                vals[b].at[pl.ds(j * IDXW, IDXW)],
                acc_sh.at[rowt.at[ck * NSUB + j]],
                ssem[b],
            ).wait()

    iota_l = lax.iota(jnp.int32, L)

    def scale(ck, b):
        vb = vals[b]
        wtc = wt.at[pl.ds(pl.multiple_of(ck * K, K), K)]

        @plsc.parallel_loop(0, K // L, unroll=1)
        def _(g):
            w16 = wtc[pl.ds(g * L, L)]
            for i in range(L):
                e = g * L + i
                wvf = jnp.broadcast_to(w16[i], (L,))
                wv = plsc.pack(wvf, wvf, format=plsc.PackFormat.INTERLEAVED)
                vb[e, pl.ds(0, 2 * L)] = vb[e, pl.ds(0, 2 * L)] * wv

    issue_gathers(0, 0)

    @pl.loop(0, NCHUNK, step=2)
    def _(ck2):
        @pl.when(ck2 >= 2)
        def _():
            wait_scatters(ck2 - 1, 1)

        issue_gathers(ck2 + 1, 1)
        wait_gathers(ck2, 0)
        scale(ck2, 0)
        issue_scatters(ck2, 0)

        wait_gathers(ck2 + 1, 1)
        scale(ck2 + 1, 1)
        wait_scatters(ck2, 0)

        @pl.when(ck2 + 2 < NCHUNK)
        def _():
            issue_gathers(ck2 + 2, 0)

        issue_scatters(ck2 + 1, 1)

    wait_scatters(NCHUNK - 1, 1)

    plsc.subcore_barrier()
    pltpu.sync_copy(acc_sh.at[pl.ds(r0, ROWS_PT)],
                    out.at[cid, pl.ds(r0, ROWS_PT)])


def kernel(x, row, col, nnz_weight, bias):
    xflat = (x.astype(jnp.bfloat16)
             .reshape(NC, HB, INDIM).transpose(0, 2, 1).reshape(NC * INDIM, HB))
    colm = col.reshape(NNZ // IDXW, IDXW)
    rowm = row.reshape(NNZ // IDXW, IDXW)
    out2 = _build_sc_expander()(xflat, colm, rowm, nnz_weight, bias)
    return out2.transpose(0, 2, 1).reshape(BATCH, OUTDIM).astype(jnp.float32)


# speedup vs baseline: 1.2602x; 1.0706x over previous
import functools

import jax
import jax.numpy as jnp
from jax import lax
from jax.experimental import pallas as pl
from jax.experimental.pallas import tpu as pltpu
from jax.experimental.pallas import tpu_sc as plsc

INDIM = 16384
OUTDIM = 16384
NNZ = 262144
BATCH = 64

NC = 2
NS = 16
L = 16

HB = BATCH // NC
K = 1024
EPT = NNZ // NS
NCHUNK = EPT // K
ROWS_PT = OUTDIM // NS
IDXW = 128
NSUB = K // IDXW
GBYTES = IDXW * HB * 4


@functools.cache
def _build_sc_expander():
    return functools.partial(
        pl.kernel,
        out_type=jax.ShapeDtypeStruct((NC, OUTDIM, HB), jnp.bfloat16),
        mesh=plsc.VectorSubcoreMesh(
            core_axis_name="c", subcore_axis_name="s",
            num_cores=NC, num_subcores=NS,
        ),
        compiler_params=pltpu.CompilerParams(
            use_tc_tiling_on_sc=False, needs_layout_passes=False
        ),
        scratch_types=[
            pltpu.VMEM_SHARED((OUTDIM, HB), jnp.bfloat16),
            pltpu.VMEM((EPT // IDXW, IDXW), jnp.int32),
            pltpu.VMEM((EPT // IDXW, IDXW), jnp.int32),
            pltpu.VMEM((EPT,), jnp.float32),
            pltpu.VMEM((K, HB), jnp.bfloat16),
            pltpu.VMEM((K, HB), jnp.bfloat16),
            pltpu.VMEM((ROWS_PT,), jnp.float32),
            pltpu.SemaphoreType.DMA,
            pltpu.SemaphoreType.DMA,
            pltpu.SemaphoreType.DMA,
            pltpu.SemaphoreType.DMA,
            pltpu.SemaphoreType.DMA,
        ],
    )(_sc_expander_body)


def _sc_expander_body(xflat, colm, rowm, w, bias, out, acc_sh, colt, rowt,
                      wt, vals0, vals1, bias_v, isem, gsem0, gsem1, ssem0,
                      ssem1):
    cid = lax.axis_index("c")
    sid = lax.axis_index("s")
    vals = (vals0, vals1)
    gsem = (gsem0, gsem1)
    ssem = (ssem0, ssem1)

    mrow0 = pl.multiple_of(sid * (EPT // IDXW), EPT // IDXW)
    e0 = pl.multiple_of(sid * EPT, EPT)
    pltpu.async_copy(colm.at[pl.ds(mrow0, EPT // IDXW)], colt, isem)
    pltpu.async_copy(rowm.at[pl.ds(mrow0, EPT // IDXW)], rowt, isem)
    pltpu.async_copy(w.at[pl.ds(e0, EPT)], wt, isem)

    r0 = pl.multiple_of(sid * ROWS_PT, ROWS_PT)
    pltpu.sync_copy(bias.at[pl.ds(r0, ROWS_PT)], bias_v)
    for rb in range(ROWS_PT // K):

        @plsc.parallel_loop(0, K // L, unroll=1)
        def _(g):
            b16 = bias_v[pl.ds(rb * K + g * L, L)]
            for i in range(L):
                o = g * L + i
                bvf = jnp.broadcast_to(b16[i], (L,))
                vals0[o, pl.ds(0, 2 * L)] = plsc.pack(
                    bvf, bvf, format=plsc.PackFormat.INTERLEAVED
                )

        pltpu.sync_copy(vals0, acc_sh.at[pl.ds(r0 + rb * K, K)])

    pltpu.make_async_copy(colm.at[pl.ds(mrow0, EPT // IDXW)], colt, isem).wait()
    pltpu.make_async_copy(rowm.at[pl.ds(mrow0, EPT // IDXW)], rowt, isem).wait()
    pltpu.make_async_copy(w.at[pl.ds(e0, EPT)], wt, isem).wait()

    cshift = cid * INDIM

    @plsc.parallel_loop(0, EPT // IDXW, unroll=2)
    def _(j):
        for kk in range(IDXW // L):
            sl = pl.ds(kk * L, L)
            colt[j, sl] = colt[j, sl] + cshift

    plsc.subcore_barrier()

    def issue_gathers(ck, b):
        for j in range(NSUB):
            pltpu.async_copy(
                xflat.at[colt.at[ck * NSUB + j]],
                vals[b].at[pl.ds(j * IDXW, IDXW)],
                gsem[b],
            )

    def wait_gathers(ck, b):
        for j in range(NSUB):
            pltpu.make_async_copy(
                xflat.at[colt.at[ck * NSUB + j]],
                vals[b].at[pl.ds(j * IDXW, IDXW)],
                gsem[b],
            ).wait()

    def issue_scatters(ck, b):
        for j in range(NSUB):
            pltpu.async_copy(
                vals[b].at[pl.ds(j * IDXW, IDXW)],
                acc_sh.at[rowt.at[ck * NSUB + j]],
                ssem[b],
                add=True,
            )

    def wait_scatters(ck, b):
        for j in range(NSUB):
            pltpu.make_async_copy(
                vals[b].at[pl.ds(j * IDXW, IDXW)],
                acc_sh.at[rowt.at[ck * NSUB + j]],
                ssem[b],
            ).wait()

    iota_l = lax.iota(jnp.int32, L)

    def scale(ck, b):
        vb = vals[b]
        wtc = wt.at[pl.ds(pl.multiple_of(ck * K, K), K)]

        @plsc.parallel_loop(0, K // L, unroll=1)
        def _(g):
            w16 = wtc[pl.ds(g * L, L)]
            for i in range(L):
                e = g * L + i
                wvf = jnp.broadcast_to(w16[i], (L,))
                wv = plsc.pack(wvf, wvf, format=plsc.PackFormat.INTERLEAVED)
                vb[e, pl.ds(0, 2 * L)] = vb[e, pl.ds(0, 2 * L)] * wv

    issue_gathers(0, 0)

    @pl.loop(0, NCHUNK, step=2)
    def _(ck2):
        @pl.when(ck2 >= 2)
        def _():
            wait_scatters(ck2 - 1, 1)

        issue_gathers(ck2 + 1, 1)
        wait_gathers(ck2, 0)
        scale(ck2, 0)
        issue_scatters(ck2, 0)

        wait_gathers(ck2 + 1, 1)
        scale(ck2 + 1, 1)
        wait_scatters(ck2, 0)

        @pl.when(ck2 + 2 < NCHUNK)
        def _():
            issue_gathers(ck2 + 2, 0)

        issue_scatters(ck2 + 1, 1)

    wait_scatters(NCHUNK - 1, 1)

    plsc.subcore_barrier()
    pltpu.sync_copy(acc_sh.at[pl.ds(r0, ROWS_PT)],
                    out.at[cid, pl.ds(r0, ROWS_PT)])


def kernel(x, row, col, nnz_weight, bias):
    xflat = (x.astype(jnp.bfloat16)
             .reshape(NC, HB, INDIM).transpose(0, 2, 1).reshape(NC * INDIM, HB))
    colm = col.reshape(NNZ // IDXW, IDXW)
    rowm = row.reshape(NNZ // IDXW, IDXW)
    out2 = _build_sc_expander()(xflat, colm, rowm, nnz_weight, bias)
    return out2.transpose(0, 2, 1).reshape(BATCH, OUTDIM).astype(jnp.float32)
